# baseline (speedup 1.0000x reference)
import functools

import jax
import jax.numpy as jnp
from jax import lax
from jax.experimental import pallas as pl
from jax.experimental.pallas import tpu as pltpu
from jax.experimental.pallas import tpu_sc as plsc

_N = 10000
_E = 320000
_D = 128
_H = 64
_C = 7

_NC = 2
_NS = 16
_NW = _NC * _NS
_EPW = _E // _NW
_CH = 80
_GS = 48
_NCH = _EPW // _CH
_RPT = _N // _NS

_SC_PARAMS = dict(
    compiler_params=pltpu.CompilerParams(needs_layout_passes=False,
                                         use_tc_tiling_on_sc=False),
)


def _sc_mesh():
    return plsc.VectorSubcoreMesh(core_axis_name="c", subcore_axis_name="s")


def _splat(vec16, e):
    return jax.lax.gather(
        vec16,
        jnp.full((16, 1), e, jnp.int32),
        jax.lax.GatherDimensionNumbers(
            offset_dims=(), collapsed_slice_dims=(0,), start_index_map=(0,)),
        (1,),
        mode=jax.lax.GatherScatterMode.PROMISE_IN_BOUNDS)


@functools.partial(
    pl.kernel,
    out_type=(
        jax.ShapeDtypeStruct((_E,), jnp.float32),
        jax.ShapeDtypeStruct((_NW, _N), jnp.float32),
        jax.ShapeDtypeStruct((_NW, _N), jnp.float32),
    ),
    mesh=_sc_mesh(),
    scratch_types=[
        pltpu.VMEM((_EPW,), jnp.int32),
        pltpu.VMEM((_EPW,), jnp.int32),
        pltpu.VMEM((_EPW,), jnp.float32),
        pltpu.VMEM((_N,), jnp.float32),
        pltpu.VMEM((_N,), jnp.float32),
        pltpu.SemaphoreType.DMA,
    ],
    **_SC_PARAMS,
)
def _sc_prep(src_hbm, dst_hbm, ewp_hbm, ew_hbm, degp_hbm, sp_hbm,
             srcv, dstv, ewv, degl, sloc, sem):
    c = lax.axis_index("c")
    s = lax.axis_index("s")
    wid = s * _NC + c
    base = wid * _EPW

    pltpu.async_copy(src_hbm.at[pl.ds(base, _EPW)], srcv, sem)
    pltpu.async_copy(dst_hbm.at[pl.ds(base, _EPW)], dstv, sem)
    pltpu.async_copy(ewp_hbm.at[pl.ds(base, _EPW)], ewv, sem)

    zero16 = jnp.zeros((16,), jnp.float32)

    def zbody(i, carry):
        for g in range(5):
            sl16 = pl.ds(i * 80 + g * 16, 16)
            degl[sl16] = zero16
            sloc[sl16] = zero16
        return carry

    lax.fori_loop(0, _N // 80, zbody, 0)

    pltpu.make_async_copy(src_hbm.at[pl.ds(base, _EPW)], srcv, sem).wait()
    pltpu.make_async_copy(dst_hbm.at[pl.ds(base, _EPW)], dstv, sem).wait()
    pltpu.make_async_copy(ewp_hbm.at[pl.ds(base, _EPW)], ewv, sem).wait()

    def chunk(i, carry):
        for g in range(5):
            sl16 = pl.ds(i * 80 + g * 16, 16)
            src16 = srcv[sl16]
            dst16 = dstv[sl16]
            w = 1.0 / (1.0 + jnp.exp(-ewv[sl16]))
            ewv[sl16] = w
            plsc.addupdate_scatter(degl, [dst16], w)
            plsc.addupdate_scatter(sloc, [src16], w, mask=dst16 == 0)
        return carry

    lax.fori_loop(0, _EPW // 80, chunk, 0)

    pltpu.sync_copy(ewv, ew_hbm.at[pl.ds(base, _EPW)])
    pltpu.sync_copy(degl, degp_hbm.at[wid])
    pltpu.sync_copy(sloc, sp_hbm.at[wid])


@functools.partial(
    pl.kernel,
    out_type=jax.ShapeDtypeStruct((_NC, _N, _H), jnp.float32),
    mesh=_sc_mesh(),
    scratch_types=[
        pltpu.VMEM((_NCH, _GS), jnp.int32),
        pltpu.VMEM((_NCH, _CH - _GS), jnp.int32),
        pltpu.VMEM((_NCH, _CH), jnp.int32),
        pltpu.VMEM((_NCH, _CH), jnp.float32),
        pltpu.VMEM((2, _CH, _H), jnp.float32),
        pltpu.VMEM((_N,), jnp.float32),
        pltpu.VMEM_SHARED((_N, _H), jnp.float32),
        pltpu.VMEM_SHARED((_N, _H), jnp.float32),
        pltpu.SemaphoreType.DMA,
        pltpu.SemaphoreType.DMA,
        pltpu.SemaphoreType.DMA,
        pltpu.SemaphoreType.DMA,
    ],
    **_SC_PARAMS,
)
def _sc_agg(srcs_hbm, srch_hbm, dst_hbm, ew_hbm, dis_hbm, hp_hbm, zero_hbm,
            out_hbm, srcs, srch, dstv, cv, rows, disv, acc, hps,
            sem_in, sem_g, sem_gh, sem_s):
    c = lax.axis_index("c")
    s = lax.axis_index("s")
    wid = s * _NC + c

    pltpu.async_copy(srcs_hbm.at[wid], srcs, sem_in)
    pltpu.async_copy(srch_hbm.at[wid], srch, sem_in)
    pltpu.async_copy(dst_hbm.at[wid], dstv, sem_in)
    pltpu.async_copy(ew_hbm.at[wid], cv, sem_in)
    pltpu.async_copy(dis_hbm, disv, sem_in)
    pltpu.sync_copy(hp_hbm.at[pl.ds(s * _RPT, _RPT)],
                    hps.at[pl.ds(s * _RPT, _RPT)])
    pltpu.sync_copy(zero_hbm.at[pl.ds(s * _RPT, _RPT)],
                    acc.at[pl.ds(s * _RPT, _RPT)])
    pltpu.make_async_copy(srcs_hbm.at[wid], srcs, sem_in).wait()
    pltpu.make_async_copy(srch_hbm.at[wid], srch, sem_in).wait()
    pltpu.make_async_copy(dst_hbm.at[wid], dstv, sem_in).wait()
    pltpu.make_async_copy(ew_hbm.at[wid], cv, sem_in).wait()
    pltpu.make_async_copy(dis_hbm, disv, sem_in).wait()

    def coef(i, carry):
        for g in range(_GS // 16):
            sl16 = pl.ds(g * 16, 16)
            cv[i, sl16] = cv[i, sl16] * plsc.load_gather(disv, [srcs[i, sl16]])
        for g in range((_CH - _GS) // 16):
            slh = pl.ds(g * 16, 16)
            sld = pl.ds(_GS + g * 16, 16)
            cv[i, sld] = cv[i, sld] * plsc.load_gather(disv, [srch[i, slh]])
        return carry

    lax.fori_loop(0, _NCH, coef, 0)

    plsc.subcore_barrier()

    def wait_g():
        pltpu.make_async_copy(hp_hbm.at[pl.ds(0, _GS)],
                              rows.at[0].at[pl.ds(0, _GS)], sem_g).wait()
        pltpu.make_async_copy(hp_hbm.at[pl.ds(0, _CH - _GS)],
                              rows.at[0].at[pl.ds(_GS, _CH - _GS)],
                              sem_gh).wait()

    def wait_s():
        pltpu.make_async_copy(hp_hbm.at[pl.ds(0, _CH)], rows.at[0],
                              sem_s).wait()

    def gather(i, buf):
        pltpu.async_copy(hps.at[srcs.at[i]],
                         rows.at[buf].at[pl.ds(0, _GS)], sem_g)
        pltpu.async_copy(hp_hbm.at[srch.at[i]],
                         rows.at[buf].at[pl.ds(_GS, _CH - _GS)], sem_gh)

    def scale(i, buf):
        for g in range(_CH // 16):
            coef16 = cv[i, pl.ds(g * 16, 16)]
            for e in range(16):
                spl = _splat(coef16, e)
                r = g * 16 + e
                for j in range(_H // 16):
                    sj = pl.ds(j * 16, 16)
                    rows[buf, r, sj] = rows[buf, r, sj] * spl

    def scatter(i, buf):
        pltpu.async_copy(rows.at[buf], acc.at[dstv.at[i]], sem_s, add=True)


    gather(0, 0)
    wait_g()
    gather(1, 1)
    scale(0, 0)
    scatter(0, 0)

    def pair(k, carry):
        i1 = 2 * k + 1
        wait_g()
        wait_s()
        gather(i1 + 1, 0)
        scale(i1, 1)
        scatter(i1, 1)
        i2 = 2 * k + 2
        wait_g()
        wait_s()
        gather(i2 + 1, 1)
        scale(i2, 0)
        scatter(i2, 0)
        return carry

    _NPAIR = (_NCH - 2) // 2
    lax.fori_loop(0, _NPAIR, pair, 0)

    wait_g()
    wait_s()
    gather(_NCH - 1, 0)
    scale(_NCH - 2, 1)
    scatter(_NCH - 2, 1)
    wait_g()
    wait_s()
    scale(_NCH - 1, 0)
    scatter(_NCH - 1, 0)
    wait_s()

    plsc.subcore_barrier()
    pltpu.sync_copy(acc.at[pl.ds(s * _RPT, _RPT)],
                    out_hbm.at[c, pl.ds(s * _RPT, _RPT)])


def _tc_mm1(x, w1):
    def body(x_ref, w_ref, o_ref):
        o_ref[...] = jnp.dot(x_ref[...], w_ref[...],
                             preferred_element_type=jnp.float32)

    return pl.pallas_call(
        body, out_shape=jax.ShapeDtypeStruct((_N, _H), jnp.float32))(x, w1)


def _tc_scal(degp, sp):
    def body(degp_ref, sp_ref, dis_ref, u2_ref):
        deg = 1.0 + jnp.sum(degp_ref[...], axis=0, keepdims=True)
        dis = lax.rsqrt(deg)
        svec = jnp.sum(sp_ref[...], axis=0, keepdims=True)
        col = lax.broadcasted_iota(jnp.int32, (1, _N), 1)
        dis0 = jnp.sum(jnp.where(col == 0, dis, 0.0))
        u = svec * dis + jnp.where(col == 0, dis, 0.0)
        dis_ref[...] = dis
        u2_ref[...] = dis0 * u

    return pl.pallas_call(
        body,
        out_shape=(
            jax.ShapeDtypeStruct((1, _N), jnp.float32),
            jax.ShapeDtypeStruct((1, _N), jnp.float32),
        ))(degp, sp)


def _tc_mid(aggp, xw, dis_col, b, w_next):
    def body(aggp_ref, xw_ref, dis_ref, b_ref, w_ref, o_ref):
        agg = aggp_ref[0] + aggp_ref[1]
        dis = dis_ref[...]
        h = jnp.maximum(dis * agg + dis * dis * xw_ref[...] + b_ref[...], 0.0)
        o_ref[...] = jnp.dot(h, w_ref[...],
                             preferred_element_type=jnp.float32)

    return pl.pallas_call(
        body, out_shape=jax.ShapeDtypeStruct((_N, w_next.shape[1]),
                                             jnp.float32))(
        aggp, xw, dis_col, b, w_next)


def _tc_fin(aggp, xw, dis_col, b2, u2, w3, b3):
    def body(aggp_ref, xw_ref, dis_ref, b2_ref, u2_ref, w3_ref, b3_ref, o_ref):
        agg = aggp_ref[0] + aggp_ref[1]
        dis = dis_ref[...]
        h2 = jnp.maximum(dis * agg + dis * dis * xw_ref[...] + b2_ref[...],
                         0.0)
        v = jnp.dot(u2_ref[...], h2, preferred_element_type=jnp.float32)
        row = jnp.dot(v, w3_ref[...],
                      preferred_element_type=jnp.float32) + b3_ref[...]
        m = jnp.max(row, axis=1, keepdims=True)
        z = row - m
        lse = jnp.log(jnp.sum(jnp.exp(z), axis=1, keepdims=True))
        o_ref[...] = z - lse

    return pl.pallas_call(
        body, out_shape=jax.ShapeDtypeStruct((1, _C), jnp.float32))(
        aggp, xw, dis_col, b2, u2, w3, b3)


def kernel(x, edge_index, edge_weight_params, W1, b1, W2, b2, W3, b3):
    src = edge_index[0]
    dst = edge_index[1]
    zeros_nh = jnp.zeros((_N, _H), jnp.float32)
    src3 = src.reshape(_NW, _NCH, _CH)
    srcs3 = src3[:, :, :_GS]
    srch3 = src3[:, :, _GS:]
    dst3 = dst.reshape(_NW, _NCH, _CH)

    ew, degp, sp = _sc_prep(src, dst, edge_weight_params)
    ew3 = ew.reshape(_NW, _NCH, _CH)
    xw1 = _tc_mm1(x, W1)
    dis_row, u2_row = _tc_scal(degp, sp)
    dis_flat = dis_row.reshape(_N)
    dis_col = dis_row.reshape(_N, 1)

    agg1 = _sc_agg(srcs3, srch3, dst3, ew3, dis_flat, xw1, zeros_nh)
    xw2 = _tc_mid(agg1, xw1, dis_col, b1.reshape(1, _H), W2)
    agg2 = _sc_agg(srcs3, srch3, dst3, ew3, dis_flat, xw2, zeros_nh)
    out = _tc_fin(agg2, xw2, dis_col, b2.reshape(1, _H), u2_row, W3,
                  b3.reshape(1, _C))
    return out.reshape(_C)


# speedup vs baseline: 1.1801x; 1.1801x over previous
import functools

import jax
import jax.numpy as jnp
from jax import lax
from jax.experimental import pallas as pl
from jax.experimental.pallas import tpu as pltpu
from jax.experimental.pallas import tpu_sc as plsc

_N = 10000
_E = 320000
_D = 128
_H = 64
_C = 7

_NC = 2
_NS = 16
_NW = _NC * _NS
_EPW = _E // _NW
_CH = 80
_GS = 48
_NCH = _EPW // _CH
_RPT = _N // _NS

_SC_PARAMS = dict(
    compiler_params=pltpu.CompilerParams(needs_layout_passes=False,
                                         use_tc_tiling_on_sc=False),
)


def _sc_mesh():
    return plsc.VectorSubcoreMesh(core_axis_name="c", subcore_axis_name="s")


def _splat(vec16, e):
    return jax.lax.gather(
        vec16,
        jnp.full((16, 1), e, jnp.int32),
        jax.lax.GatherDimensionNumbers(
            offset_dims=(), collapsed_slice_dims=(0,), start_index_map=(0,)),
        (1,),
        mode=jax.lax.GatherScatterMode.PROMISE_IN_BOUNDS)


@functools.partial(
    pl.kernel,
    out_type=(
        jax.ShapeDtypeStruct((_E,), jnp.float32),
        jax.ShapeDtypeStruct((_NW, _N), jnp.float32),
        jax.ShapeDtypeStruct((_NW, _N), jnp.float32),
    ),
    mesh=_sc_mesh(),
    scratch_types=[
        pltpu.VMEM((_EPW,), jnp.int32),
        pltpu.VMEM((_EPW,), jnp.int32),
        pltpu.VMEM((_EPW,), jnp.float32),
        pltpu.VMEM((_N,), jnp.float32),
        pltpu.VMEM((_N,), jnp.float32),
        pltpu.SemaphoreType.DMA,
    ],
    **_SC_PARAMS,
)
def _sc_prep(src_hbm, dst_hbm, ewp_hbm, ew_hbm, degp_hbm, sp_hbm,
             srcv, dstv, ewv, degl, sloc, sem):
    c = lax.axis_index("c")
    s = lax.axis_index("s")
    wid = s * _NC + c
    base = wid * _EPW

    pltpu.async_copy(src_hbm.at[pl.ds(base, _EPW)], srcv, sem)
    pltpu.async_copy(dst_hbm.at[pl.ds(base, _EPW)], dstv, sem)
    pltpu.async_copy(ewp_hbm.at[pl.ds(base, _EPW)], ewv, sem)

    zero16 = jnp.zeros((16,), jnp.float32)

    def zbody(i, carry):
        for g in range(5):
            sl16 = pl.ds(i * 80 + g * 16, 16)
            degl[sl16] = zero16
            sloc[sl16] = zero16
        return carry

    lax.fori_loop(0, _N // 80, zbody, 0)

    pltpu.make_async_copy(src_hbm.at[pl.ds(base, _EPW)], srcv, sem).wait()
    pltpu.make_async_copy(dst_hbm.at[pl.ds(base, _EPW)], dstv, sem).wait()
    pltpu.make_async_copy(ewp_hbm.at[pl.ds(base, _EPW)], ewv, sem).wait()

    def chunk(i, carry):
        for g in range(5):
            sl16 = pl.ds(i * 80 + g * 16, 16)
            src16 = srcv[sl16]
            dst16 = dstv[sl16]
            w = 1.0 / (1.0 + jnp.exp(-ewv[sl16]))
            ewv[sl16] = w
            plsc.addupdate_scatter(degl, [dst16], w)
            plsc.addupdate_scatter(sloc, [src16], w, mask=dst16 == 0)
        return carry

    lax.fori_loop(0, _EPW // 80, chunk, 0)

    pltpu.sync_copy(ewv, ew_hbm.at[pl.ds(base, _EPW)])
    pltpu.sync_copy(degl, degp_hbm.at[wid])
    pltpu.sync_copy(sloc, sp_hbm.at[wid])


@functools.partial(
    pl.kernel,
    out_type=jax.ShapeDtypeStruct((_NC, _N, _H), jnp.float32),
    mesh=_sc_mesh(),
    scratch_types=[
        pltpu.VMEM((_NCH, _GS), jnp.int32),
        pltpu.VMEM((_NCH, _CH - _GS), jnp.int32),
        pltpu.VMEM((_NCH, _CH), jnp.int32),
        pltpu.VMEM((_NCH, _CH), jnp.float32),
        pltpu.VMEM((2, _CH, _H), jnp.float32),
        pltpu.VMEM((_N,), jnp.float32),
        pltpu.VMEM_SHARED((_N, _H), jnp.float32),
        pltpu.VMEM_SHARED((_N, _H), jnp.float32),
        pltpu.SemaphoreType.DMA,
        pltpu.SemaphoreType.DMA,
        pltpu.SemaphoreType.DMA,
        pltpu.SemaphoreType.DMA,
    ],
    **_SC_PARAMS,
)
def _sc_agg(srcs_hbm, srch_hbm, dst_hbm, ew_hbm, dis_hbm, hp_hbm, zero_hbm,
            out_hbm, srcs, srch, dstv, cv, rows, disv, acc, hps,
            sem_in, sem_g, sem_gh, sem_s):
    c = lax.axis_index("c")
    s = lax.axis_index("s")
    wid = s * _NC + c

    pltpu.async_copy(srcs_hbm.at[wid], srcs, sem_in)
    pltpu.async_copy(srch_hbm.at[wid], srch, sem_in)
    pltpu.async_copy(dst_hbm.at[wid], dstv, sem_in)
    pltpu.async_copy(ew_hbm.at[wid], cv, sem_in)
    pltpu.async_copy(dis_hbm, disv, sem_in)
    pltpu.sync_copy(hp_hbm.at[pl.ds(s * _RPT, _RPT)],
                    hps.at[pl.ds(s * _RPT, _RPT)])
    pltpu.sync_copy(zero_hbm.at[pl.ds(s * _RPT, _RPT)],
                    acc.at[pl.ds(s * _RPT, _RPT)])
    pltpu.make_async_copy(srcs_hbm.at[wid], srcs, sem_in).wait()
    pltpu.make_async_copy(srch_hbm.at[wid], srch, sem_in).wait()
    pltpu.make_async_copy(dst_hbm.at[wid], dstv, sem_in).wait()
    pltpu.make_async_copy(ew_hbm.at[wid], cv, sem_in).wait()
    pltpu.make_async_copy(dis_hbm, disv, sem_in).wait()

    def coef(i, carry):
        for g in range(_GS // 16):
            sl16 = pl.ds(g * 16, 16)
            cv[i, sl16] = cv[i, sl16] * plsc.load_gather(disv, [srcs[i, sl16]])
        for g in range((_CH - _GS) // 16):
            slh = pl.ds(g * 16, 16)
            sld = pl.ds(_GS + g * 16, 16)
            cv[i, sld] = cv[i, sld] * plsc.load_gather(disv, [srch[i, slh]])
        return carry

    lax.fori_loop(0, _NCH, coef, 0)

    plsc.subcore_barrier()

    def wait_g():
        pltpu.make_async_copy(hp_hbm.at[pl.ds(0, _GS)],
                              rows.at[0].at[pl.ds(0, _GS)], sem_g).wait()
        pltpu.make_async_copy(hp_hbm.at[pl.ds(0, _CH - _GS)],
                              rows.at[0].at[pl.ds(_GS, _CH - _GS)],
                              sem_gh).wait()

    def wait_s():
        pltpu.make_async_copy(hp_hbm.at[pl.ds(0, _CH)], rows.at[0],
                              sem_s).wait()

    def gather(i, buf):
        pltpu.async_copy(hps.at[srcs.at[i]],
                         rows.at[buf].at[pl.ds(0, _GS)], sem_g)
        pltpu.async_copy(hps.at[srch.at[i]],
                         rows.at[buf].at[pl.ds(_GS, _CH - _GS)], sem_gh)

    def scale(i, buf):
        for g in range(_CH // 16):
            coef16 = cv[i, pl.ds(g * 16, 16)]
            for e in range(16):
                spl = _splat(coef16, e)
                r = g * 16 + e
                for j in range(_H // 16):
                    sj = pl.ds(j * 16, 16)
                    rows[buf, r, sj] = rows[buf, r, sj] * spl

    def scatter(i, buf):
        pltpu.async_copy(rows.at[buf], acc.at[dstv.at[i]], sem_s, add=True)


    gather(0, 0)
    wait_g()
    gather(1, 1)
    scale(0, 0)
    scatter(0, 0)

    def pair(k, carry):
        i1 = 2 * k + 1
        wait_g()
        wait_s()
        gather(i1 + 1, 0)
        scale(i1, 1)
        scatter(i1, 1)
        i2 = 2 * k + 2
        wait_g()
        wait_s()
        gather(i2 + 1, 1)
        scale(i2, 0)
        scatter(i2, 0)
        return carry

    _NPAIR = (_NCH - 2) // 2
    lax.fori_loop(0, _NPAIR, pair, 0)

    wait_g()
    wait_s()
    gather(_NCH - 1, 0)
    scale(_NCH - 2, 1)
    scatter(_NCH - 2, 1)
    wait_g()
    wait_s()
    scale(_NCH - 1, 0)
    scatter(_NCH - 1, 0)
    wait_s()

    plsc.subcore_barrier()
    pltpu.sync_copy(acc.at[pl.ds(s * _RPT, _RPT)],
                    out_hbm.at[c, pl.ds(s * _RPT, _RPT)])


def _tc_mm1(x, w1):
    def body(x_ref, w_ref, o_ref):
        o_ref[...] = jnp.dot(x_ref[...], w_ref[...],
                             preferred_element_type=jnp.float32)

    return pl.pallas_call(
        body, out_shape=jax.ShapeDtypeStruct((_N, _H), jnp.float32))(x, w1)


def _tc_scal(degp, sp):
    def body(degp_ref, sp_ref, dis_ref, u2_ref):
        deg = 1.0 + jnp.sum(degp_ref[...], axis=0, keepdims=True)
        dis = lax.rsqrt(deg)
        svec = jnp.sum(sp_ref[...], axis=0, keepdims=True)
        col = lax.broadcasted_iota(jnp.int32, (1, _N), 1)
        dis0 = jnp.sum(jnp.where(col == 0, dis, 0.0))
        u = svec * dis + jnp.where(col == 0, dis, 0.0)
        dis_ref[...] = dis
        u2_ref[...] = dis0 * u

    return pl.pallas_call(
        body,
        out_shape=(
            jax.ShapeDtypeStruct((1, _N), jnp.float32),
            jax.ShapeDtypeStruct((1, _N), jnp.float32),
        ))(degp, sp)


def _tc_mid(aggp, xw, dis_col, b, w_next):
    def body(aggp_ref, xw_ref, dis_ref, b_ref, w_ref, o_ref):
        agg = aggp_ref[0] + aggp_ref[1]
        dis = dis_ref[...]
        h = jnp.maximum(dis * agg + dis * dis * xw_ref[...] + b_ref[...], 0.0)
        o_ref[...] = jnp.dot(h, w_ref[...],
                             preferred_element_type=jnp.float32)

    return pl.pallas_call(
        body, out_shape=jax.ShapeDtypeStruct((_N, w_next.shape[1]),
                                             jnp.float32))(
        aggp, xw, dis_col, b, w_next)


def _tc_fin(aggp, xw, dis_col, b2, u2, w3, b3):
    def body(aggp_ref, xw_ref, dis_ref, b2_ref, u2_ref, w3_ref, b3_ref, o_ref):
        agg = aggp_ref[0] + aggp_ref[1]
        dis = dis_ref[...]
        h2 = jnp.maximum(dis * agg + dis * dis * xw_ref[...] + b2_ref[...],
                         0.0)
        v = jnp.dot(u2_ref[...], h2, preferred_element_type=jnp.float32)
        row = jnp.dot(v, w3_ref[...],
                      preferred_element_type=jnp.float32) + b3_ref[...]
        m = jnp.max(row, axis=1, keepdims=True)
        z = row - m
        lse = jnp.log(jnp.sum(jnp.exp(z), axis=1, keepdims=True))
        o_ref[...] = z - lse

    return pl.pallas_call(
        body, out_shape=jax.ShapeDtypeStruct((1, _C), jnp.float32))(
        aggp, xw, dis_col, b2, u2, w3, b3)


def kernel(x, edge_index, edge_weight_params, W1, b1, W2, b2, W3, b3):
    src = edge_index[0]
    dst = edge_index[1]
    zeros_nh = jnp.zeros((_N, _H), jnp.float32)
    src3 = src.reshape(_NW, _NCH, _CH)
    srcs3 = src3[:, :, :_GS]
    srch3 = src3[:, :, _GS:]
    dst3 = dst.reshape(_NW, _NCH, _CH)

    ew, degp, sp = _sc_prep(src, dst, edge_weight_params)
    ew3 = ew.reshape(_NW, _NCH, _CH)
    xw1 = _tc_mm1(x, W1)
    dis_row, u2_row = _tc_scal(degp, sp)
    dis_flat = dis_row.reshape(_N)
    dis_col = dis_row.reshape(_N, 1)

    agg1 = _sc_agg(srcs3, srch3, dst3, ew3, dis_flat, xw1, zeros_nh)
    xw2 = _tc_mid(agg1, xw1, dis_col, b1.reshape(1, _H), W2)
    agg2 = _sc_agg(srcs3, srch3, dst3, ew3, dis_flat, xw2, zeros_nh)
    out = _tc_fin(agg2, xw2, dis_col, b2.reshape(1, _H), u2_row, W3,
                  b3.reshape(1, _C))
    return out.reshape(_C)


# speedup vs baseline: 1.2038x; 1.0200x over previous
import functools

import jax
import jax.numpy as jnp
from jax import lax
from jax.experimental import pallas as pl
from jax.experimental.pallas import tpu as pltpu
from jax.experimental.pallas import tpu_sc as plsc

_N = 10000
_E = 320000
_D = 128
_H = 64
_C = 7

_NC = 2
_NS = 16
_NW = _NC * _NS
_EPW = _E // _NW
_CH = 80
_GS = 48
_NCH = _EPW // _CH
_RPT = _N // _NS

_SC_PARAMS = dict(
    compiler_params=pltpu.CompilerParams(needs_layout_passes=False,
                                         use_tc_tiling_on_sc=False),
)


def _sc_mesh():
    return plsc.VectorSubcoreMesh(core_axis_name="c", subcore_axis_name="s")


def _splat(vec16, e):
    return jax.lax.gather(
        vec16,
        jnp.full((16, 1), e, jnp.int32),
        jax.lax.GatherDimensionNumbers(
            offset_dims=(), collapsed_slice_dims=(0,), start_index_map=(0,)),
        (1,),
        mode=jax.lax.GatherScatterMode.PROMISE_IN_BOUNDS)


@functools.partial(
    pl.kernel,
    out_type=(
        jax.ShapeDtypeStruct((_E,), jnp.float32),
        jax.ShapeDtypeStruct((_NW, _N), jnp.float32),
        jax.ShapeDtypeStruct((_NW, _N), jnp.float32),
    ),
    mesh=_sc_mesh(),
    scratch_types=[
        pltpu.VMEM((_EPW,), jnp.int32),
        pltpu.VMEM((_EPW,), jnp.int32),
        pltpu.VMEM((_EPW,), jnp.float32),
        pltpu.VMEM((_N,), jnp.float32),
        pltpu.VMEM((_N,), jnp.float32),
        pltpu.SemaphoreType.DMA,
    ],
    **_SC_PARAMS,
)
def _sc_prep(src_hbm, dst_hbm, ewp_hbm, ew_hbm, degp_hbm, sp_hbm,
             srcv, dstv, ewv, degl, sloc, sem):
    c = lax.axis_index("c")
    s = lax.axis_index("s")
    wid = s * _NC + c
    base = wid * _EPW

    pltpu.async_copy(src_hbm.at[pl.ds(base, _EPW)], srcv, sem)
    pltpu.async_copy(dst_hbm.at[pl.ds(base, _EPW)], dstv, sem)
    pltpu.async_copy(ewp_hbm.at[pl.ds(base, _EPW)], ewv, sem)

    zero16 = jnp.zeros((16,), jnp.float32)

    def zbody(i, carry):
        for g in range(5):
            sl16 = pl.ds(i * 80 + g * 16, 16)
            degl[sl16] = zero16
            sloc[sl16] = zero16
        return carry

    lax.fori_loop(0, _N // 80, zbody, 0)

    pltpu.make_async_copy(src_hbm.at[pl.ds(base, _EPW)], srcv, sem).wait()
    pltpu.make_async_copy(dst_hbm.at[pl.ds(base, _EPW)], dstv, sem).wait()
    pltpu.make_async_copy(ewp_hbm.at[pl.ds(base, _EPW)], ewv, sem).wait()

    def chunk(i, carry):
        for g in range(5):
            sl16 = pl.ds(i * 80 + g * 16, 16)
            src16 = srcv[sl16]
            dst16 = dstv[sl16]
            w = 1.0 / (1.0 + jnp.exp(-ewv[sl16]))
            ewv[sl16] = w
            plsc.addupdate_scatter(degl, [dst16], w)
            plsc.addupdate_scatter(sloc, [src16], w, mask=dst16 == 0)
        return carry

    lax.fori_loop(0, _EPW // 80, chunk, 0)

    pltpu.sync_copy(ewv, ew_hbm.at[pl.ds(base, _EPW)])
    pltpu.sync_copy(degl, degp_hbm.at[wid])
    pltpu.sync_copy(sloc, sp_hbm.at[wid])


@functools.partial(
    pl.kernel,
    out_type=jax.ShapeDtypeStruct((_NC, _N, _H), jnp.float32),
    mesh=_sc_mesh(),
    scratch_types=[
        pltpu.VMEM((_NCH, _CH), jnp.int32),
        pltpu.VMEM((_NCH, _CH), jnp.int32),
        pltpu.VMEM((_NCH, _CH), jnp.float32),
        pltpu.VMEM((2, _CH, _H), jnp.float32),
        pltpu.VMEM((_N,), jnp.float32),
        pltpu.VMEM_SHARED((_N, _H), jnp.float32),
        pltpu.VMEM_SHARED((_N, _H), jnp.float32),
        pltpu.SemaphoreType.DMA,
        pltpu.SemaphoreType.DMA,
        pltpu.SemaphoreType.DMA,
    ],
    **_SC_PARAMS,
)
def _sc_agg(src_hbm, dst_hbm, ew_hbm, dis_hbm, hp_hbm, zero_hbm,
            out_hbm, srcv, dstv, cv, rows, disv, acc, hps,
            sem_in, sem_g, sem_s):
    c = lax.axis_index("c")
    s = lax.axis_index("s")
    wid = s * _NC + c

    pltpu.async_copy(src_hbm.at[wid], srcv, sem_in)
    pltpu.async_copy(dst_hbm.at[wid], dstv, sem_in)
    pltpu.async_copy(ew_hbm.at[wid], cv, sem_in)
    pltpu.async_copy(dis_hbm, disv, sem_in)
    pltpu.sync_copy(hp_hbm.at[pl.ds(s * _RPT, _RPT)],
                    hps.at[pl.ds(s * _RPT, _RPT)])
    pltpu.sync_copy(zero_hbm.at[pl.ds(s * _RPT, _RPT)],
                    acc.at[pl.ds(s * _RPT, _RPT)])
    pltpu.make_async_copy(src_hbm.at[wid], srcv, sem_in).wait()
    pltpu.make_async_copy(dst_hbm.at[wid], dstv, sem_in).wait()
    pltpu.make_async_copy(ew_hbm.at[wid], cv, sem_in).wait()
    pltpu.make_async_copy(dis_hbm, disv, sem_in).wait()

    def coef(i, carry):
        for g in range(_CH // 16):
            sl16 = pl.ds(g * 16, 16)
            cv[i, sl16] = cv[i, sl16] * plsc.load_gather(disv, [srcv[i, sl16]])
        return carry

    lax.fori_loop(0, _NCH, coef, 0)

    plsc.subcore_barrier()

    def wait_g():
        pltpu.make_async_copy(hp_hbm.at[pl.ds(0, _CH)], rows.at[0],
                              sem_g).wait()

    def wait_s():
        pltpu.make_async_copy(hp_hbm.at[pl.ds(0, _CH)], rows.at[0],
                              sem_s).wait()

    def gather(i, buf):
        pltpu.async_copy(hps.at[srcv.at[i]], rows.at[buf], sem_g)

    def scale(i, buf):
        for g in range(_CH // 16):
            coef16 = cv[i, pl.ds(g * 16, 16)]
            for e in range(16):
                spl = _splat(coef16, e)
                r = g * 16 + e
                for j in range(_H // 16):
                    sj = pl.ds(j * 16, 16)
                    rows[buf, r, sj] = rows[buf, r, sj] * spl

    def scatter(i, buf):
        pltpu.async_copy(rows.at[buf], acc.at[dstv.at[i]], sem_s, add=True)


    gather(0, 0)
    wait_g()
    gather(1, 1)
    scale(0, 0)
    scatter(0, 0)

    def pair(k, carry):
        i1 = 2 * k + 1
        wait_g()
        wait_s()
        gather(i1 + 1, 0)
        scale(i1, 1)
        scatter(i1, 1)
        i2 = 2 * k + 2
        wait_g()
        wait_s()
        gather(i2 + 1, 1)
        scale(i2, 0)
        scatter(i2, 0)
        return carry

    _NPAIR = (_NCH - 2) // 2
    lax.fori_loop(0, _NPAIR, pair, 0)

    wait_g()
    wait_s()
    gather(_NCH - 1, 0)
    scale(_NCH - 2, 1)
    scatter(_NCH - 2, 1)
    wait_g()
    wait_s()
    scale(_NCH - 1, 0)
    scatter(_NCH - 1, 0)
    wait_s()

    plsc.subcore_barrier()
    pltpu.sync_copy(acc.at[pl.ds(s * _RPT, _RPT)],
                    out_hbm.at[c, pl.ds(s * _RPT, _RPT)])


def _tc_mm1(x, w1):
    def body(x_ref, w_ref, o_ref):
        o_ref[...] = jnp.dot(x_ref[...], w_ref[...],
                             preferred_element_type=jnp.float32)

    return pl.pallas_call(
        body, out_shape=jax.ShapeDtypeStruct((_N, _H), jnp.float32))(x, w1)


def _tc_scal(degp, sp):
    def body(degp_ref, sp_ref, dis_ref, u2_ref):
        deg = 1.0 + jnp.sum(degp_ref[...], axis=0, keepdims=True)
        dis = lax.rsqrt(deg)
        svec = jnp.sum(sp_ref[...], axis=0, keepdims=True)
        col = lax.broadcasted_iota(jnp.int32, (1, _N), 1)
        dis0 = jnp.sum(jnp.where(col == 0, dis, 0.0))
        u = svec * dis + jnp.where(col == 0, dis, 0.0)
        dis_ref[...] = dis
        u2_ref[...] = dis0 * u

    return pl.pallas_call(
        body,
        out_shape=(
            jax.ShapeDtypeStruct((1, _N), jnp.float32),
            jax.ShapeDtypeStruct((1, _N), jnp.float32),
        ))(degp, sp)


def _tc_mid(aggp, xw, dis_col, b, w_next):
    def body(aggp_ref, xw_ref, dis_ref, b_ref, w_ref, o_ref):
        agg = aggp_ref[0] + aggp_ref[1]
        dis = dis_ref[...]
        h = jnp.maximum(dis * agg + dis * dis * xw_ref[...] + b_ref[...], 0.0)
        o_ref[...] = jnp.dot(h, w_ref[...],
                             preferred_element_type=jnp.float32)

    return pl.pallas_call(
        body, out_shape=jax.ShapeDtypeStruct((_N, w_next.shape[1]),
                                             jnp.float32))(
        aggp, xw, dis_col, b, w_next)


def _tc_fin(aggp, xw, dis_col, b2, u2, w3, b3):
    def body(aggp_ref, xw_ref, dis_ref, b2_ref, u2_ref, w3_ref, b3_ref, o_ref):
        agg = aggp_ref[0] + aggp_ref[1]
        dis = dis_ref[...]
        h2 = jnp.maximum(dis * agg + dis * dis * xw_ref[...] + b2_ref[...],
                         0.0)
        v = jnp.dot(u2_ref[...], h2, preferred_element_type=jnp.float32)
        row = jnp.dot(v, w3_ref[...],
                      preferred_element_type=jnp.float32) + b3_ref[...]
        m = jnp.max(row, axis=1, keepdims=True)
        z = row - m
        lse = jnp.log(jnp.sum(jnp.exp(z), axis=1, keepdims=True))
        o_ref[...] = z - lse

    return pl.pallas_call(
        body, out_shape=jax.ShapeDtypeStruct((1, _C), jnp.float32))(
        aggp, xw, dis_col, b2, u2, w3, b3)


def kernel(x, edge_index, edge_weight_params, W1, b1, W2, b2, W3, b3):
    src = edge_index[0]
    dst = edge_index[1]
    zeros_nh = jnp.zeros((_N, _H), jnp.float32)
    src3 = src.reshape(_NW, _NCH, _CH)
    dst3 = dst.reshape(_NW, _NCH, _CH)

    ew, degp, sp = _sc_prep(src, dst, edge_weight_params)
    ew3 = ew.reshape(_NW, _NCH, _CH)
    xw1 = _tc_mm1(x, W1)
    dis_row, u2_row = _tc_scal(degp, sp)
    dis_flat = dis_row.reshape(_N)
    dis_col = dis_row.reshape(_N, 1)

    agg1 = _sc_agg(src3, dst3, ew3, dis_flat, xw1, zeros_nh)
    xw2 = _tc_mid(agg1, xw1, dis_col, b1.reshape(1, _H), W2)
    agg2 = _sc_agg(src3, dst3, ew3, dis_flat, xw2, zeros_nh)
    out = _tc_fin(agg2, xw2, dis_col, b2.reshape(1, _H), u2_row, W3,
                  b3.reshape(1, _C))
    return out.reshape(_C)
